# baseline (speedup 1.0000x reference)
import functools

import jax
import jax.numpy as jnp
from jax import lax
from jax.experimental import pallas as pl
from jax.experimental.pallas import tpu as pltpu
from jax.experimental.pallas import tpu_sc as plsc

T = 2048
D = 1024
DFF = 2048
E = 8
K = 2
ALPHA = 0.001

B = 128
NB = (K * T) // B + E
NBB = NB * B
LANES = 128
NEG = -1e30


def _gate_body(x_ref, wg_ref, bias_ref, probs_ref, ti_ref, tw_ref):
    s = lax.dot_general(x_ref[...], wg_ref[...], (((1,), (1,)), ((), ())),
                        preferred_element_type=jnp.float32)
    col = lax.broadcasted_iota(jnp.int32, s.shape, 1)
    valid = col < E
    s = jnp.where(valid, s, NEG)
    m = jnp.max(s, axis=1, keepdims=True)
    p = jnp.exp(s - m)
    p = jnp.where(valid, p, 0.0)
    probs = p / jnp.sum(p, axis=1, keepdims=True)
    biased = probs + bias_ref[...][0:1, :]
    biased = jnp.where(valid, biased, NEG)
    m1 = jnp.max(biased, axis=1, keepdims=True)
    i1 = jnp.min(jnp.where(biased == m1, col, LANES), axis=1, keepdims=True)
    b2 = jnp.where(col == i1, NEG, biased)
    m2 = jnp.max(b2, axis=1, keepdims=True)
    i2 = jnp.min(jnp.where(b2 == m2, col, LANES), axis=1, keepdims=True)
    w1 = jnp.sum(jnp.where(col == i1, probs, 0.0), axis=1, keepdims=True)
    w2 = jnp.sum(jnp.where(col == i2, probs, 0.0), axis=1, keepdims=True)
    probs_ref[...] = probs
    ti_ref[...] = jnp.where(col == 0, i1, jnp.where(col == 1, i2, 0))
    tw_ref[...] = jnp.where(col == 0, w1, jnp.where(col == 1, w2, 0.0))


def _gate(x, Wg, gate_bias):
    bt = 256
    wg_pad = jnp.zeros((LANES, D), jnp.float32).at[:E].set(Wg)
    bias_pad = jnp.zeros((8, LANES), jnp.float32).at[0, :E].set(gate_bias)
    return pl.pallas_call(
        _gate_body,
        grid=(T // bt,),
        in_specs=[
            pl.BlockSpec((bt, D), lambda i: (i, 0)),
            pl.BlockSpec((LANES, D), lambda i: (0, 0)),
            pl.BlockSpec((8, LANES), lambda i: (0, 0)),
        ],
        out_specs=[
            pl.BlockSpec((bt, LANES), lambda i: (i, 0)),
            pl.BlockSpec((bt, LANES), lambda i: (i, 0)),
            pl.BlockSpec((bt, LANES), lambda i: (i, 0)),
        ],
        out_shape=[
            jax.ShapeDtypeStruct((T, LANES), jnp.float32),
            jax.ShapeDtypeStruct((T, LANES), jnp.int32),
            jax.ShapeDtypeStruct((T, LANES), jnp.float32),
        ],
    )(x, wg_pad, bias_pad)


def _make_sc_gather(n_rows, n_src_rows, chunk):
    info = plsc.get_sparse_core_info()
    nw = info.num_cores * info.num_subcores
    per_w = n_rows // nw
    assert per_w % chunk == 0 and chunk % 8 == 0
    n_iter = per_w // chunk
    mesh = plsc.VectorSubcoreMesh(core_axis_name="c", subcore_axis_name="s")

    @functools.partial(
        pl.kernel, mesh=mesh,
        out_type=jax.ShapeDtypeStruct((n_rows, D), jnp.float32),
        scratch_types=[
            pltpu.VMEM((chunk,), jnp.int32),
            pltpu.VMEM((chunk, D), jnp.float32),
            pltpu.SemaphoreType.DMA,
        ],
    )
    def k(table_hbm, idx_hbm, out_hbm, idx_v, rows_v, sem):
        wid = lax.axis_index("s") * info.num_cores + lax.axis_index("c")
        base = wid * per_w
        for it in range(n_iter):
            off = base + it * chunk
            pltpu.sync_copy(idx_hbm.at[pl.ds(off, chunk)], idx_v)
            pltpu.async_copy(table_hbm.at[idx_v], rows_v, sem).wait()
            pltpu.sync_copy(rows_v, out_hbm.at[pl.ds(off, chunk)])

    return k


def _routed_body(be_ref, xs_ref, wfc_ref, bfc_ref, wproj_ref, bproj_ref,
                 w_ref, out_ref):
    h = lax.dot_general(xs_ref[...], wfc_ref[...][0], (((1,), (1,)), ((), ())),
                        preferred_element_type=jnp.float32)
    h = h + bfc_ref[...][0]
    h = h * jax.nn.sigmoid(h)
    o = lax.dot_general(h, wproj_ref[...][0], (((1,), (1,)), ((), ())),
                        preferred_element_type=jnp.float32)
    o = o + bproj_ref[...][0]
    out_ref[...] = o * w_ref[...][:, 0:1]


def _routed_mlp(xs, Wfc, bfc, Wproj, bproj, wpadb, be):
    grid_spec = pltpu.PrefetchScalarGridSpec(
        num_scalar_prefetch=1,
        grid=(NB,),
        in_specs=[
            pl.BlockSpec((B, D), lambda i, be: (i, 0)),
            pl.BlockSpec((1, DFF, D), lambda i, be: (be[i], 0, 0)),
            pl.BlockSpec((1, 1, DFF), lambda i, be: (be[i], 0, 0)),
            pl.BlockSpec((1, D, DFF), lambda i, be: (be[i], 0, 0)),
            pl.BlockSpec((1, 1, D), lambda i, be: (be[i], 0, 0)),
            pl.BlockSpec((B, LANES), lambda i, be: (i, 0)),
        ],
        out_specs=pl.BlockSpec((B, D), lambda i, be: (i, 0)),
    )
    return pl.pallas_call(
        _routed_body,
        grid_spec=grid_spec,
        out_shape=jax.ShapeDtypeStruct((NBB, D), jnp.float32),
        compiler_params=pltpu.CompilerParams(
            dimension_semantics=("arbitrary",)),
    )(be, xs, Wfc, bfc[:, None, :], Wproj, bproj[:, None, :], wpadb)


def _combine_body(x_ref, wsfc_ref, bsfc_ref, wsproj_ref, bsproj_ref,
                  g0_ref, g1_ref, y_ref):
    h = lax.dot_general(x_ref[...], wsfc_ref[...], (((1,), (1,)), ((), ())),
                        preferred_element_type=jnp.float32)
    h = h + bsfc_ref[...][0][None, :]
    h = h * jax.nn.sigmoid(h)
    z = lax.dot_general(h, wsproj_ref[...], (((1,), (1,)), ((), ())),
                        preferred_element_type=jnp.float32)
    z = z + bsproj_ref[...][0][None, :]
    y_ref[...] = z + g0_ref[...] + g1_ref[...]


def _combine(x, Ws_fc, bs_fc, Ws_proj, bs_proj, g0, g1):
    bt = 256
    return pl.pallas_call(
        _combine_body,
        grid=(T // bt,),
        in_specs=[
            pl.BlockSpec((bt, D), lambda i: (i, 0)),
            pl.BlockSpec((DFF, D), lambda i: (0, 0)),
            pl.BlockSpec((1, DFF), lambda i: (0, 0)),
            pl.BlockSpec((D, DFF), lambda i: (0, 0)),
            pl.BlockSpec((1, D), lambda i: (0, 0)),
            pl.BlockSpec((bt, D), lambda i: (i, 0)),
            pl.BlockSpec((bt, D), lambda i: (i, 0)),
        ],
        out_specs=pl.BlockSpec((bt, D), lambda i: (i, 0)),
        out_shape=jax.ShapeDtypeStruct((T, D), jnp.float32),
    )(x, Ws_fc, bs_fc[None, :], Ws_proj, bs_proj[None, :], g0, g1)


def kernel(x, Wg, gate_bias, Wfc, bfc, Wproj, bproj, Ws_fc, bs_fc, Ws_proj, bs_proj):
    probs_pad, ti, tw = _gate(x, Wg, gate_bias)
    probs = probs_pad[:, :E]
    i1, i2 = ti[:, 0], ti[:, 1]
    w1, w2 = tw[:, 0], tw[:, 1]

    e_f = jnp.stack([i1, i2], axis=1).reshape(-1)
    oh = (e_f[:, None] == jnp.arange(E)[None, :]).astype(jnp.int32)
    csum = jnp.cumsum(oh, axis=0)
    counts = csum[-1]
    rank = jnp.take_along_axis(csum, e_f[:, None], axis=1)[:, 0] - 1
    nb_e = (counts + B - 1) // B
    blk_start = jnp.cumsum(nb_e) - nb_e
    p = blk_start[e_f] * B + rank
    tok = jnp.repeat(jnp.arange(T, dtype=jnp.int32), K)
    src = jnp.zeros((NBB,), jnp.int32).at[p].set(tok)
    w_f = jnp.stack([w1, w2], axis=1).reshape(-1)
    wpad = jnp.zeros((NBB,), jnp.float32).at[p].set(w_f)
    wpadb = jnp.broadcast_to(wpad[:, None], (NBB, LANES))
    bidx = jnp.arange(NB)
    be = jnp.sum((bidx[:, None] >= blk_start[None, :]).astype(jnp.int32),
                 axis=1) - 1
    be = jnp.clip(be, 0, E - 1).astype(jnp.int32)
    pos0, pos1 = p[0::K], p[1::K]

    xs = _make_sc_gather(NBB, T, 80)(x, src)
    contrib = _routed_mlp(xs, Wfc, bfc, Wproj, bproj, wpadb, be)
    g0 = _make_sc_gather(T, NBB, 64)(contrib, pos0.astype(jnp.int32))
    g1 = _make_sc_gather(T, NBB, 64)(contrib, pos1.astype(jnp.int32))
    y = _combine(x, Ws_fc, bs_fc, Ws_proj, bs_proj, g0, g1)

    expert_probs = probs.mean(axis=0)
    f_i = counts.astype(jnp.float32) * E / (K * T + 1e-06)
    load_balance_loss = ALPHA * jnp.sum(f_i * expert_probs)
    return (y, load_balance_loss)


# speedup vs baseline: 1.1808x; 1.1808x over previous
import functools

import jax
import jax.numpy as jnp
from jax import lax
from jax.experimental import pallas as pl
from jax.experimental.pallas import tpu as pltpu
from jax.experimental.pallas import tpu_sc as plsc

T = 2048
D = 1024
DFF = 2048
E = 8
K = 2
ALPHA = 0.001

B = 128
NB = (K * T) // B + E
NBB = NB * B
LANES = 128
NEG = -1e30


def _gate_body(x_ref, wg_ref, bias_ref, probs_ref, ti_ref, tw_ref):
    s = lax.dot_general(x_ref[...], wg_ref[...], (((1,), (1,)), ((), ())),
                        preferred_element_type=jnp.float32)
    col = lax.broadcasted_iota(jnp.int32, s.shape, 1)
    valid = col < E
    s = jnp.where(valid, s, NEG)
    m = jnp.max(s, axis=1, keepdims=True)
    p = jnp.exp(s - m)
    p = jnp.where(valid, p, 0.0)
    probs = p / jnp.sum(p, axis=1, keepdims=True)
    biased = probs + bias_ref[...][0:1, :]
    biased = jnp.where(valid, biased, NEG)
    m1 = jnp.max(biased, axis=1, keepdims=True)
    i1 = jnp.min(jnp.where(biased == m1, col, LANES), axis=1, keepdims=True)
    b2 = jnp.where(col == i1, NEG, biased)
    m2 = jnp.max(b2, axis=1, keepdims=True)
    i2 = jnp.min(jnp.where(b2 == m2, col, LANES), axis=1, keepdims=True)
    w1 = jnp.sum(jnp.where(col == i1, probs, 0.0), axis=1, keepdims=True)
    w2 = jnp.sum(jnp.where(col == i2, probs, 0.0), axis=1, keepdims=True)
    colE = col[:, :E]
    probs_ref[...] = probs[:, :E]
    ti_ref[...] = jnp.where(colE == 0, i1, jnp.where(colE == 1, i2, 0))
    tw_ref[...] = jnp.where(colE == 0, w1, jnp.where(colE == 1, w2, 0.0))


def _gate(x, Wg, gate_bias):
    bt = 256
    wg_pad = jnp.zeros((LANES, D), jnp.float32).at[:E].set(Wg)
    bias_pad = jnp.zeros((8, LANES), jnp.float32).at[0, :E].set(gate_bias)
    return pl.pallas_call(
        _gate_body,
        grid=(T // bt,),
        in_specs=[
            pl.BlockSpec((bt, D), lambda i: (i, 0)),
            pl.BlockSpec((LANES, D), lambda i: (0, 0)),
            pl.BlockSpec((8, LANES), lambda i: (0, 0)),
        ],
        out_specs=[
            pl.BlockSpec((bt, E), lambda i: (i, 0)),
            pl.BlockSpec((bt, E), lambda i: (i, 0)),
            pl.BlockSpec((bt, E), lambda i: (i, 0)),
        ],
        out_shape=[
            jax.ShapeDtypeStruct((T, E), jnp.float32),
            jax.ShapeDtypeStruct((T, E), jnp.int32),
            jax.ShapeDtypeStruct((T, E), jnp.float32),
        ],
    )(x, wg_pad, bias_pad)


def _make_sc_gather(n_rows, chunk):
    info = plsc.get_sparse_core_info()
    nw = info.num_cores * info.num_subcores
    per_w = n_rows // nw
    assert n_rows % nw == 0 and per_w % chunk == 0 and chunk % 8 == 0
    n_iter = per_w // chunk
    nbuf = min(2, n_iter)
    mesh = plsc.VectorSubcoreMesh(core_axis_name="c", subcore_axis_name="s")

    @functools.partial(
        pl.kernel, mesh=mesh,
        out_type=jax.ShapeDtypeStruct((n_rows, D), jnp.float32),
        scratch_types=[
            pltpu.VMEM((per_w,), jnp.int32),
            pltpu.VMEM((nbuf, chunk, D), jnp.float32),
            pltpu.SemaphoreType.DMA,
            pltpu.SemaphoreType.DMA,
            pltpu.SemaphoreType.DMA,
            pltpu.SemaphoreType.DMA,
        ],
    )
    def k(table_hbm, idx_hbm, out_hbm, idx_v, rows_v, gs0, gs1, ss0, ss1):
        wid = lax.axis_index("s") * info.num_cores + lax.axis_index("c")
        base = wid * per_w
        pltpu.sync_copy(idx_hbm.at[pl.ds(base, per_w)], idx_v)
        gsems = [gs0, gs1]
        ssems = [ss0, ss1]
        gat = [None, None]
        scat = [None, None]
        for i in range(n_iter + 1):
            b = i % nbuf
            if i < n_iter:
                if scat[b] is not None:
                    scat[b].wait()
                    scat[b] = None
                gat[b] = pltpu.async_copy(
                    table_hbm.at[idx_v.at[pl.ds(i * chunk, chunk)]],
                    rows_v.at[b], gsems[b])
            if i >= 1:
                pb = (i - 1) % nbuf
                gat[pb].wait()
                scat[pb] = pltpu.async_copy(
                    rows_v.at[pb],
                    out_hbm.at[pl.ds(base + (i - 1) * chunk, chunk)],
                    ssems[pb])
        for b in range(nbuf):
            if scat[b] is not None:
                scat[b].wait()

    return k


def _routed_body(be_ref, xs_ref, wfc_ref, bfc_ref, wproj_ref, bproj_ref,
                 w_ref, out_ref):
    h = lax.dot_general(xs_ref[...], wfc_ref[...][0], (((1,), (1,)), ((), ())),
                        preferred_element_type=jnp.float32)
    h = h + bfc_ref[...][0]
    h = h * jax.nn.sigmoid(h)
    o = lax.dot_general(h, wproj_ref[...][0], (((1,), (1,)), ((), ())),
                        preferred_element_type=jnp.float32)
    o = o + bproj_ref[...][0]
    out_ref[...] = o * w_ref[...][:, 0:1]


def _routed_mlp(xs, Wfc, bfc, Wproj, bproj, wpadb, be):
    grid_spec = pltpu.PrefetchScalarGridSpec(
        num_scalar_prefetch=1,
        grid=(NB,),
        in_specs=[
            pl.BlockSpec((B, D), lambda i, be: (i, 0)),
            pl.BlockSpec((1, DFF, D), lambda i, be: (be[i], 0, 0)),
            pl.BlockSpec((1, 1, DFF), lambda i, be: (be[i], 0, 0)),
            pl.BlockSpec((1, D, DFF), lambda i, be: (be[i], 0, 0)),
            pl.BlockSpec((1, 1, D), lambda i, be: (be[i], 0, 0)),
            pl.BlockSpec((B, E), lambda i, be: (i, 0)),
        ],
        out_specs=pl.BlockSpec((B, D), lambda i, be: (i, 0)),
    )
    return pl.pallas_call(
        _routed_body,
        grid_spec=grid_spec,
        out_shape=jax.ShapeDtypeStruct((NBB, D), jnp.float32),
        compiler_params=pltpu.CompilerParams(
            dimension_semantics=("arbitrary",)),
    )(be, xs, Wfc, bfc[:, None, :], Wproj, bproj[:, None, :], wpadb)


def _shared_body(x_ref, wsfc_ref, bsfc_ref, wsproj_ref, bsproj_ref, z_ref):
    h = lax.dot_general(x_ref[...], wsfc_ref[...], (((1,), (1,)), ((), ())),
                        preferred_element_type=jnp.float32)
    h = h + bsfc_ref[...][0][None, :]
    h = h * jax.nn.sigmoid(h)
    z = lax.dot_general(h, wsproj_ref[...], (((1,), (1,)), ((), ())),
                        preferred_element_type=jnp.float32)
    z_ref[...] = z + bsproj_ref[...][0][None, :]


def _shared_mlp(x, Ws_fc, bs_fc, Ws_proj, bs_proj):
    bt = 256
    return pl.pallas_call(
        _shared_body,
        grid=(T // bt,),
        in_specs=[
            pl.BlockSpec((bt, D), lambda i: (i, 0)),
            pl.BlockSpec((DFF, D), lambda i: (0, 0)),
            pl.BlockSpec((1, DFF), lambda i: (0, 0)),
            pl.BlockSpec((D, DFF), lambda i: (0, 0)),
            pl.BlockSpec((1, D), lambda i: (0, 0)),
        ],
        out_specs=pl.BlockSpec((bt, D), lambda i: (i, 0)),
        out_shape=jax.ShapeDtypeStruct((T, D), jnp.float32),
    )(x, Ws_fc, bs_fc[None, :], Ws_proj, bs_proj[None, :])


def _add3_body(a_ref, b_ref, c_ref, y_ref):
    y_ref[...] = a_ref[...] + b_ref[...] + c_ref[...]


def _add3(a, b, c):
    bt = 512
    return pl.pallas_call(
        _add3_body,
        grid=(T // bt,),
        in_specs=[pl.BlockSpec((bt, D), lambda i: (i, 0))] * 3,
        out_specs=pl.BlockSpec((bt, D), lambda i: (i, 0)),
        out_shape=jax.ShapeDtypeStruct((T, D), jnp.float32),
    )(a, b, c)


def kernel(x, Wg, gate_bias, Wfc, bfc, Wproj, bproj, Ws_fc, bs_fc, Ws_proj, bs_proj):
    probs, ti, tw = _gate(x, Wg, gate_bias)
    i1, i2 = ti[:, 0], ti[:, 1]
    w1, w2 = tw[:, 0], tw[:, 1]

    e_f = jnp.stack([i1, i2], axis=1).reshape(-1)
    oh = (e_f[:, None] == jnp.arange(E)[None, :]).astype(jnp.int32)
    csum = jnp.cumsum(oh, axis=0)
    counts = csum[-1]
    rank = jnp.take_along_axis(csum, e_f[:, None], axis=1)[:, 0] - 1
    nb_e = (counts + B - 1) // B
    blk_start = jnp.cumsum(nb_e) - nb_e
    p = blk_start[e_f] * B + rank
    tok = jnp.repeat(jnp.arange(T, dtype=jnp.int32), K)
    src = (jnp.arange(NBB, dtype=jnp.int32) % T).at[p].set(tok)
    w_f = jnp.stack([w1, w2], axis=1).reshape(-1)
    wpad = jnp.zeros((NBB,), jnp.float32).at[p].set(w_f)
    wpadb = jnp.broadcast_to(wpad[:, None], (NBB, E))
    bidx = jnp.arange(NB)
    be = jnp.sum((bidx[:, None] >= blk_start[None, :]).astype(jnp.int32),
                 axis=1) - 1
    be = jnp.clip(be, 0, E - 1).astype(jnp.int32)
    pos0, pos1 = p[0::K], p[1::K]

    z = _shared_mlp(x, Ws_fc, bs_fc, Ws_proj, bs_proj)
    xs = _make_sc_gather(NBB, 40)(x, src)
    contrib = _routed_mlp(xs, Wfc, bfc, Wproj, bproj, wpadb, be)
    g0 = _make_sc_gather(T, 64)(contrib, pos0.astype(jnp.int32))
    g1 = _make_sc_gather(T, 64)(contrib, pos1.astype(jnp.int32))
    y = _add3(z, g0, g1)

    expert_probs = probs.mean(axis=0)
    f_i = counts.astype(jnp.float32) * E / (K * T + 1e-06)
    load_balance_loss = ALPHA * jnp.sum(f_i * expert_probs)
    return (y, load_balance_loss)


# speedup vs baseline: 1.2022x; 1.0181x over previous
import functools

import jax
import jax.numpy as jnp
from jax import lax
from jax.experimental import pallas as pl
from jax.experimental.pallas import tpu as pltpu
from jax.experimental.pallas import tpu_sc as plsc

T = 2048
D = 1024
DFF = 2048
E = 8
K = 2
ALPHA = 0.001

B = 128
NB = (K * T) // B + E
NBB = NB * B
LANES = 128
NEG = -1e30


def _gate_body(x_ref, wg_ref, bias_ref, probs_ref, ti_ref, tw_ref):
    s = lax.dot_general(x_ref[...], wg_ref[...], (((1,), (1,)), ((), ())),
                        preferred_element_type=jnp.float32)
    col = lax.broadcasted_iota(jnp.int32, s.shape, 1)
    valid = col < E
    s = jnp.where(valid, s, NEG)
    m = jnp.max(s, axis=1, keepdims=True)
    p = jnp.exp(s - m)
    p = jnp.where(valid, p, 0.0)
    probs = p / jnp.sum(p, axis=1, keepdims=True)
    biased = probs + bias_ref[...][0:1, :]
    biased = jnp.where(valid, biased, NEG)
    m1 = jnp.max(biased, axis=1, keepdims=True)
    i1 = jnp.min(jnp.where(biased == m1, col, LANES), axis=1, keepdims=True)
    b2 = jnp.where(col == i1, NEG, biased)
    m2 = jnp.max(b2, axis=1, keepdims=True)
    i2 = jnp.min(jnp.where(b2 == m2, col, LANES), axis=1, keepdims=True)
    w1 = jnp.sum(jnp.where(col == i1, probs, 0.0), axis=1, keepdims=True)
    w2 = jnp.sum(jnp.where(col == i2, probs, 0.0), axis=1, keepdims=True)
    colE = col[:, :E]
    probs_ref[...] = probs[:, :E]
    ti_ref[...] = jnp.where(colE == 0, i1, jnp.where(colE == 1, i2, 0))
    tw_ref[...] = jnp.where(colE == 0, w1, jnp.where(colE == 1, w2, 0.0))


def _gate(x, Wg, gate_bias):
    bt = 256
    wg_pad = jnp.zeros((LANES, D), jnp.float32).at[:E].set(Wg)
    bias_pad = jnp.zeros((8, LANES), jnp.float32).at[0, :E].set(gate_bias)
    return pl.pallas_call(
        _gate_body,
        grid=(T // bt,),
        in_specs=[
            pl.BlockSpec((bt, D), lambda i: (i, 0)),
            pl.BlockSpec((LANES, D), lambda i: (0, 0)),
            pl.BlockSpec((8, LANES), lambda i: (0, 0)),
        ],
        out_specs=[
            pl.BlockSpec((bt, E), lambda i: (i, 0)),
            pl.BlockSpec((bt, E), lambda i: (i, 0)),
            pl.BlockSpec((bt, E), lambda i: (i, 0)),
        ],
        out_shape=[
            jax.ShapeDtypeStruct((T, E), jnp.float32),
            jax.ShapeDtypeStruct((T, E), jnp.int32),
            jax.ShapeDtypeStruct((T, E), jnp.float32),
        ],
    )(x, wg_pad, bias_pad)


def _make_sc_gather(n_rows, chunk):
    info = plsc.get_sparse_core_info()
    nw = info.num_cores * info.num_subcores
    per_w = n_rows // nw
    assert n_rows % nw == 0 and per_w % chunk == 0 and chunk % 8 == 0
    n_iter = per_w // chunk
    nbuf = min(2, n_iter)
    mesh = plsc.VectorSubcoreMesh(core_axis_name="c", subcore_axis_name="s")

    @functools.partial(
        pl.kernel, mesh=mesh,
        out_type=jax.ShapeDtypeStruct((n_rows, D), jnp.float32),
        scratch_types=[
            pltpu.VMEM((per_w,), jnp.int32),
            pltpu.VMEM((nbuf, chunk, D), jnp.float32),
            pltpu.SemaphoreType.DMA,
            pltpu.SemaphoreType.DMA,
            pltpu.SemaphoreType.DMA,
            pltpu.SemaphoreType.DMA,
        ],
    )
    def k(table_hbm, idx_hbm, out_hbm, idx_v, rows_v, gs0, gs1, ss0, ss1):
        wid = lax.axis_index("s") * info.num_cores + lax.axis_index("c")
        base = wid * per_w
        pltpu.sync_copy(idx_hbm.at[pl.ds(base, per_w)], idx_v)
        gsems = [gs0, gs1]
        ssems = [ss0, ss1]
        gat = [None, None]
        scat = [None, None]
        for i in range(n_iter + 1):
            b = i % nbuf
            if i < n_iter:
                if scat[b] is not None:
                    scat[b].wait()
                    scat[b] = None
                gat[b] = pltpu.async_copy(
                    table_hbm.at[idx_v.at[pl.ds(i * chunk, chunk)]],
                    rows_v.at[b], gsems[b])
            if i >= 1:
                pb = (i - 1) % nbuf
                gat[pb].wait()
                scat[pb] = pltpu.async_copy(
                    rows_v.at[pb],
                    out_hbm.at[pl.ds(base + (i - 1) * chunk, chunk)],
                    ssems[pb])
        for b in range(nbuf):
            if scat[b] is not None:
                scat[b].wait()

    return k


def _routed_body(be_ref, xs_ref, wfc_ref, bfc_ref, wproj_ref, bproj_ref,
                 out_ref):
    h = lax.dot_general(xs_ref[...], wfc_ref[...][0], (((1,), (1,)), ((), ())),
                        preferred_element_type=jnp.float32)
    h = h + bfc_ref[...][0]
    h = h * jax.nn.sigmoid(h)
    o = lax.dot_general(h, wproj_ref[...][0], (((1,), (1,)), ((), ())),
                        preferred_element_type=jnp.float32)
    out_ref[...] = o + bproj_ref[...][0]


def _routed_mlp(xs, Wfc, bfc, Wproj, bproj, be):
    grid_spec = pltpu.PrefetchScalarGridSpec(
        num_scalar_prefetch=1,
        grid=(NB,),
        in_specs=[
            pl.BlockSpec((B, D), lambda i, be: (i, 0)),
            pl.BlockSpec((1, DFF, D), lambda i, be: (be[i], 0, 0)),
            pl.BlockSpec((1, 1, DFF), lambda i, be: (be[i], 0, 0)),
            pl.BlockSpec((1, D, DFF), lambda i, be: (be[i], 0, 0)),
            pl.BlockSpec((1, 1, D), lambda i, be: (be[i], 0, 0)),
        ],
        out_specs=pl.BlockSpec((B, D), lambda i, be: (i, 0)),
    )
    return pl.pallas_call(
        _routed_body,
        grid_spec=grid_spec,
        out_shape=jax.ShapeDtypeStruct((NBB, D), jnp.float32),
        compiler_params=pltpu.CompilerParams(
            dimension_semantics=("arbitrary",),
            vmem_limit_bytes=100 * 1024 * 1024),
    )(be, xs, Wfc, bfc[:, None, :], Wproj, bproj[:, None, :])


def _shared_body(x_ref, wsfc_ref, bsfc_ref, wsproj_ref, bsproj_ref, z_ref):
    h = lax.dot_general(x_ref[...], wsfc_ref[...], (((1,), (1,)), ((), ())),
                        preferred_element_type=jnp.float32)
    h = h + bsfc_ref[...][0][None, :]
    h = h * jax.nn.sigmoid(h)
    z = lax.dot_general(h, wsproj_ref[...], (((1,), (1,)), ((), ())),
                        preferred_element_type=jnp.float32)
    z_ref[...] = z + bsproj_ref[...][0][None, :]


def _shared_mlp(x, Ws_fc, bs_fc, Ws_proj, bs_proj):
    bt = 256
    return pl.pallas_call(
        _shared_body,
        grid=(T // bt,),
        in_specs=[
            pl.BlockSpec((bt, D), lambda i: (i, 0)),
            pl.BlockSpec((DFF, D), lambda i: (0, 0)),
            pl.BlockSpec((1, DFF), lambda i: (0, 0)),
            pl.BlockSpec((D, DFF), lambda i: (0, 0)),
            pl.BlockSpec((1, D), lambda i: (0, 0)),
        ],
        out_specs=pl.BlockSpec((bt, D), lambda i: (i, 0)),
        out_shape=jax.ShapeDtypeStruct((T, D), jnp.float32),
    )(x, Ws_fc, bs_fc[None, :], Ws_proj, bs_proj[None, :])


def _add3_body(z_ref, g0_ref, g1_ref, tw_ref, y_ref):
    w = tw_ref[...]
    y_ref[...] = (z_ref[...] + w[:, 0:1] * g0_ref[...]
                  + w[:, 1:2] * g1_ref[...])


def _add3(z, g0, g1, tw):
    bt = 512
    return pl.pallas_call(
        _add3_body,
        grid=(T // bt,),
        in_specs=[pl.BlockSpec((bt, D), lambda i: (i, 0))] * 3
        + [pl.BlockSpec((bt, E), lambda i: (i, 0))],
        out_specs=pl.BlockSpec((bt, D), lambda i: (i, 0)),
        out_shape=jax.ShapeDtypeStruct((T, D), jnp.float32),
    )(z, g0, g1, tw)


def kernel(x, Wg, gate_bias, Wfc, bfc, Wproj, bproj, Ws_fc, bs_fc, Ws_proj, bs_proj):
    probs, ti, tw = _gate(x, Wg, gate_bias)
    i1, i2 = ti[:, 0], ti[:, 1]
    w1, w2 = tw[:, 0], tw[:, 1]

    e_f = jnp.stack([i1, i2], axis=1).reshape(-1)
    oh = (e_f[:, None] == jnp.arange(E)[None, :]).astype(jnp.int32)
    csum = jnp.cumsum(oh, axis=0)
    counts = csum[-1]
    rank = jnp.take_along_axis(csum, e_f[:, None], axis=1)[:, 0] - 1
    nb_e = (counts + B - 1) // B
    blk_start = jnp.cumsum(nb_e) - nb_e
    p = blk_start[e_f] * B + rank
    tok = jnp.repeat(jnp.arange(T, dtype=jnp.int32), K)
    src = (jnp.arange(NBB, dtype=jnp.int32) % T).at[p].set(
        tok, unique_indices=True)
    bidx = jnp.arange(NB)
    be = jnp.sum((bidx[:, None] >= blk_start[None, :]).astype(jnp.int32),
                 axis=1) - 1
    be = jnp.clip(be, 0, E - 1).astype(jnp.int32)
    pos0, pos1 = p[0::K], p[1::K]

    z = _shared_mlp(x, Ws_fc, bs_fc, Ws_proj, bs_proj)
    xs = _make_sc_gather(NBB, 40)(x, src)
    contrib = _routed_mlp(xs, Wfc, bfc, Wproj, bproj, be)
    g0 = _make_sc_gather(T, 64)(contrib, pos0.astype(jnp.int32))
    g1 = _make_sc_gather(T, 64)(contrib, pos1.astype(jnp.int32))
    y = _add3(z, g0, g1, tw)

    expert_probs = probs.mean(axis=0)
    f_i = counts.astype(jnp.float32) * E / (K * T + 1e-06)
    load_balance_loss = ALPHA * jnp.sum(f_i * expert_probs)
    return (y, load_balance_loss)


# speedup vs baseline: 1.4751x; 1.2270x over previous
import functools

import jax
import jax.numpy as jnp
from jax import lax
from jax.experimental import pallas as pl
from jax.experimental.pallas import tpu as pltpu
from jax.experimental.pallas import tpu_sc as plsc

T = 2048
D = 1024
DFF = 2048
E = 8
K = 2
ALPHA = 0.001

B = 256
NB = (K * T) // B + E
NBB = NB * B
LANES = 128
NEG = -1e30


def _gate_body(x_ref, wg_ref, bias_ref, probs_ref, ti_ref, tw_ref):
    s = lax.dot_general(x_ref[...], wg_ref[...], (((1,), (1,)), ((), ())),
                        preferred_element_type=jnp.float32)
    col = lax.broadcasted_iota(jnp.int32, s.shape, 1)
    valid = col < E
    s = jnp.where(valid, s, NEG)
    m = jnp.max(s, axis=1, keepdims=True)
    p = jnp.exp(s - m)
    p = jnp.where(valid, p, 0.0)
    probs = p / jnp.sum(p, axis=1, keepdims=True)
    biased = probs + bias_ref[...][0:1, :]
    biased = jnp.where(valid, biased, NEG)
    m1 = jnp.max(biased, axis=1, keepdims=True)
    i1 = jnp.min(jnp.where(biased == m1, col, LANES), axis=1, keepdims=True)
    b2 = jnp.where(col == i1, NEG, biased)
    m2 = jnp.max(b2, axis=1, keepdims=True)
    i2 = jnp.min(jnp.where(b2 == m2, col, LANES), axis=1, keepdims=True)
    w1 = jnp.sum(jnp.where(col == i1, probs, 0.0), axis=1, keepdims=True)
    w2 = jnp.sum(jnp.where(col == i2, probs, 0.0), axis=1, keepdims=True)
    colE = col[:, :E]
    probs_ref[...] = probs[:, :E]
    ti_ref[...] = jnp.where(colE == 0, i1, jnp.where(colE == 1, i2, 0))
    tw_ref[...] = jnp.where(colE == 0, w1, jnp.where(colE == 1, w2, 0.0))


def _gate(x, Wg, gate_bias):
    bt = 256
    wg_pad = jnp.zeros((LANES, D), jnp.float32).at[:E].set(Wg)
    bias_pad = jnp.zeros((8, LANES), jnp.float32).at[0, :E].set(gate_bias)
    return pl.pallas_call(
        _gate_body,
        grid=(T // bt,),
        in_specs=[
            pl.BlockSpec((bt, D), lambda i: (i, 0)),
            pl.BlockSpec((LANES, D), lambda i: (0, 0)),
            pl.BlockSpec((8, LANES), lambda i: (0, 0)),
        ],
        out_specs=[
            pl.BlockSpec((bt, E), lambda i: (i, 0)),
            pl.BlockSpec((bt, E), lambda i: (i, 0)),
            pl.BlockSpec((bt, E), lambda i: (i, 0)),
        ],
        out_shape=[
            jax.ShapeDtypeStruct((T, E), jnp.float32),
            jax.ShapeDtypeStruct((T, E), jnp.int32),
            jax.ShapeDtypeStruct((T, E), jnp.float32),
        ],
    )(x, wg_pad, bias_pad)


def _make_sc_gather(n_rows, chunk):
    info = plsc.get_sparse_core_info()
    nw = info.num_cores * info.num_subcores
    per_w = n_rows // nw
    assert n_rows % nw == 0 and per_w % chunk == 0 and chunk % 8 == 0
    n_iter = per_w // chunk
    nbuf = min(2, n_iter)
    mesh = plsc.VectorSubcoreMesh(core_axis_name="c", subcore_axis_name="s")

    @functools.partial(
        pl.kernel, mesh=mesh,
        out_type=jax.ShapeDtypeStruct((n_rows, D), jnp.float32),
        scratch_types=[
            pltpu.VMEM((per_w,), jnp.int32),
            pltpu.VMEM((nbuf, chunk, D), jnp.float32),
            pltpu.SemaphoreType.DMA,
            pltpu.SemaphoreType.DMA,
            pltpu.SemaphoreType.DMA,
            pltpu.SemaphoreType.DMA,
        ],
    )
    def k(table_hbm, idx_hbm, out_hbm, idx_v, rows_v, gs0, gs1, ss0, ss1):
        wid = lax.axis_index("s") * info.num_cores + lax.axis_index("c")
        base = wid * per_w
        pltpu.sync_copy(idx_hbm.at[pl.ds(base, per_w)], idx_v)
        gsems = [gs0, gs1]
        ssems = [ss0, ss1]
        gat = [None, None]
        scat = [None, None]
        for i in range(n_iter + 1):
            b = i % nbuf
            if i < n_iter:
                if scat[b] is not None:
                    scat[b].wait()
                    scat[b] = None
                gat[b] = pltpu.async_copy(
                    table_hbm.at[idx_v.at[pl.ds(i * chunk, chunk)]],
                    rows_v.at[b], gsems[b])
            if i >= 1:
                pb = (i - 1) % nbuf
                gat[pb].wait()
                scat[pb] = pltpu.async_copy(
                    rows_v.at[pb],
                    out_hbm.at[pl.ds(base + (i - 1) * chunk, chunk)],
                    ssems[pb])
        for b in range(nbuf):
            if scat[b] is not None:
                scat[b].wait()

    return k


def _routed_body(be_ref, xs_ref, wfc_ref, bfc_ref, wproj_ref, bproj_ref,
                 out_ref):
    h = lax.dot_general(xs_ref[...], wfc_ref[...][0], (((1,), (1,)), ((), ())),
                        preferred_element_type=jnp.float32)
    h = h + bfc_ref[...][0]
    h = h * jax.nn.sigmoid(h)
    o = lax.dot_general(h, wproj_ref[...][0], (((1,), (1,)), ((), ())),
                        preferred_element_type=jnp.float32)
    out_ref[...] = o + bproj_ref[...][0]


def _routed_mlp(xs, Wfc, bfc, Wproj, bproj, be):
    grid_spec = pltpu.PrefetchScalarGridSpec(
        num_scalar_prefetch=1,
        grid=(NB,),
        in_specs=[
            pl.BlockSpec((B, D), lambda i, be: (i, 0)),
            pl.BlockSpec((1, DFF, D), lambda i, be: (be[i], 0, 0)),
            pl.BlockSpec((1, 1, DFF), lambda i, be: (be[i], 0, 0)),
            pl.BlockSpec((1, D, DFF), lambda i, be: (be[i], 0, 0)),
            pl.BlockSpec((1, 1, D), lambda i, be: (be[i], 0, 0)),
        ],
        out_specs=pl.BlockSpec((B, D), lambda i, be: (i, 0)),
    )
    return pl.pallas_call(
        _routed_body,
        grid_spec=grid_spec,
        out_shape=jax.ShapeDtypeStruct((NBB, D), jnp.float32),
        compiler_params=pltpu.CompilerParams(
            dimension_semantics=("arbitrary",),
            vmem_limit_bytes=100 * 1024 * 1024),
    )(be, xs, Wfc, bfc[:, None, :], Wproj, bproj[:, None, :])


def _shared_body(x_ref, wsfc_ref, bsfc_ref, wsproj_ref, bsproj_ref, z_ref):
    h = lax.dot_general(x_ref[...], wsfc_ref[...], (((1,), (1,)), ((), ())),
                        preferred_element_type=jnp.float32)
    h = h + bsfc_ref[...][0][None, :]
    h = h * jax.nn.sigmoid(h)
    z = lax.dot_general(h, wsproj_ref[...], (((1,), (1,)), ((), ())),
                        preferred_element_type=jnp.float32)
    z_ref[...] = z + bsproj_ref[...][0][None, :]


def _shared_mlp(x, Ws_fc, bs_fc, Ws_proj, bs_proj):
    bt = 256
    return pl.pallas_call(
        _shared_body,
        grid=(T // bt,),
        in_specs=[
            pl.BlockSpec((bt, D), lambda i: (i, 0)),
            pl.BlockSpec((DFF, D), lambda i: (0, 0)),
            pl.BlockSpec((1, DFF), lambda i: (0, 0)),
            pl.BlockSpec((D, DFF), lambda i: (0, 0)),
            pl.BlockSpec((1, D), lambda i: (0, 0)),
        ],
        out_specs=pl.BlockSpec((bt, D), lambda i: (i, 0)),
        out_shape=jax.ShapeDtypeStruct((T, D), jnp.float32),
    )(x, Ws_fc, bs_fc[None, :], Ws_proj, bs_proj[None, :])


def _add3_body(z_ref, g0_ref, g1_ref, tw_ref, y_ref):
    w = tw_ref[...]
    y_ref[...] = (z_ref[...] + w[:, 0:1] * g0_ref[...]
                  + w[:, 1:2] * g1_ref[...])


def _add3(z, g0, g1, tw):
    bt = 512
    return pl.pallas_call(
        _add3_body,
        grid=(T // bt,),
        in_specs=[pl.BlockSpec((bt, D), lambda i: (i, 0))] * 3
        + [pl.BlockSpec((bt, E), lambda i: (i, 0))],
        out_specs=pl.BlockSpec((bt, D), lambda i: (i, 0)),
        out_shape=jax.ShapeDtypeStruct((T, D), jnp.float32),
    )(z, g0, g1, tw)


def kernel(x, Wg, gate_bias, Wfc, bfc, Wproj, bproj, Ws_fc, bs_fc, Ws_proj, bs_proj):
    probs, ti, tw = _gate(x, Wg, gate_bias)
    i1, i2 = ti[:, 0], ti[:, 1]
    w1, w2 = tw[:, 0], tw[:, 1]

    e_f = jnp.stack([i1, i2], axis=1).reshape(-1)
    oh = (e_f[:, None] == jnp.arange(E)[None, :]).astype(jnp.int32)
    csum = jnp.cumsum(oh, axis=0)
    counts = csum[-1]
    rank = jnp.take_along_axis(csum, e_f[:, None], axis=1)[:, 0] - 1
    nb_e = (counts + B - 1) // B
    blk_start = jnp.cumsum(nb_e) - nb_e
    p = blk_start[e_f] * B + rank
    tok = jnp.repeat(jnp.arange(T, dtype=jnp.int32), K)
    src = (jnp.arange(NBB, dtype=jnp.int32) % T).at[p].set(
        tok, unique_indices=True)
    bidx = jnp.arange(NB)
    be = jnp.sum((bidx[:, None] >= blk_start[None, :]).astype(jnp.int32),
                 axis=1) - 1
    be = jnp.clip(be, 0, E - 1).astype(jnp.int32)
    pos0, pos1 = p[0::K], p[1::K]

    z = _shared_mlp(x, Ws_fc, bs_fc, Ws_proj, bs_proj)
    xs = _make_sc_gather(NBB, 48)(x, src)
    contrib = _routed_mlp(xs, Wfc, bfc, Wproj, bproj, be)
    g0 = _make_sc_gather(T, 64)(contrib, pos0.astype(jnp.int32))
    g1 = _make_sc_gather(T, 64)(contrib, pos1.astype(jnp.int32))
    y = _add3(z, g0, g1, tw)

    expert_probs = probs.mean(axis=0)
    f_i = counts.astype(jnp.float32) * E / (K * T + 1e-06)
    load_balance_loss = ALPHA * jnp.sum(f_i * expert_probs)
    return (y, load_balance_loss)
